# baseline (speedup 1.0000x reference)
import jax
import jax.numpy as jnp
from jax.experimental import pallas as pl

N = 50000
E = 800000
G = 128
H = 4
C = 64
NEU = 64


def _conv(x, ea, src, dst, p):
    n = x.shape[0]
    q = (x @ p['Wq'] + p['bq']).reshape(n, H, C)
    kmat = (x @ p['Wk'] + p['bk']).reshape(n, H, C)
    v = (x @ p['Wv'] + p['bv']).reshape(n, H, C)
    e = (ea @ p['We']).reshape(-1, H, C)
    k_j = kmat[src] + e
    logits = (q[dst] * k_j).sum(-1) / jnp.sqrt(float(C))
    m = jax.ops.segment_max(logits, dst, num_segments=n)
    m = jnp.where(jnp.isfinite(m), m, 0.0)
    ex = jnp.exp(logits - m[dst])
    denom = jax.ops.segment_sum(ex, dst, num_segments=n)
    alpha = ex / (denom[dst] + 1e-16)
    msg = (v[src] + e) * alpha[..., None]
    out = jax.ops.segment_sum(msg, dst, num_segments=n).reshape(n, H * C)
    return out + x @ p['Wskip'] + p['bskip']


def kernel(x, edge_index, edge_attr, batch, params):
    src, dst = edge_index[0], edge_index[1]
    perm = jnp.argsort(dst)
    src = src[perm]
    dst = dst[perm]
    ea_s = edge_attr[perm]
    h = x @ params['embed_n_W'] + params['embed_n_b']
    ea = jax.nn.elu(ea_s @ params['embed_e_W'] + params['embed_e_b'], alpha=0.2)
    for lp in params['layers']:
        h = _conv(h, ea, src, dst, lp)
        h = jax.nn.elu(h @ lp['Wlin'] + lp['blin'])
        mean = h.mean(0)
        var = h.var(0)
        h = lp['gamma'] * (h - mean) / jnp.sqrt(var + 1e-5) + lp['beta']
        h = jax.nn.relu(h)
    sums = jax.ops.segment_sum(h, batch, num_segments=G)
    cnt = jax.ops.segment_sum(jnp.ones((h.shape[0],), jnp.float32), batch, num_segments=G)
    pooled = sums / jnp.clip(cnt, 1.0)[:, None]
    return pooled @ params['lin3_W'] + params['lin3_b']


# speedup vs baseline: 5.2687x; 5.2687x over previous
import functools

import jax
import jax.numpy as jnp
from jax import lax
from jax.experimental import pallas as pl
from jax.experimental.pallas import tpu as pltpu, tpu_sc as plsc

N = 50000
E = 800000
G = 128
H = 4
C = 64
NEU = 64

NC, NS = 2, 16
NW = NC * NS
NPB = 64
NBUCKET = 800
BPW = NBUCKET // NW
NP = NBUCKET * NPB
B = 64
EPAD = E + 2 * B
BN = 1024
BE = 2000


def _sc_edge_kernel():
    mesh = plsc.VectorSubcoreMesh(core_axis_name="c", subcore_axis_name="s")
    f32 = jnp.float32
    out_type = [
        jax.ShapeDtypeStruct((NP, 128), f32),
        jax.ShapeDtypeStruct((NP, 128), f32),
        jax.ShapeDtypeStruct((NP, 128), f32),
        jax.ShapeDtypeStruct((NP, 128), f32),
        jax.ShapeDtypeStruct((EPAD // 8, 128), f32),
        jax.ShapeDtypeStruct((NP, 128), f32),
    ]
    scratch = [
        pltpu.VMEM((B,), jnp.int32),
        pltpu.VMEM((B,), jnp.int32),
        pltpu.VMEM((B,), jnp.int32),
        pltpu.VMEM((B, 128), f32),
        pltpu.VMEM((B, 128), f32),
        pltpu.VMEM((B, 128), f32),
        pltpu.VMEM((B, 128), f32),
        pltpu.VMEM((B, 128), f32),
        pltpu.VMEM((B, 128), f32),
        pltpu.VMEM((B, 128), f32),
        pltpu.VMEM((B, 128), f32),
        pltpu.VMEM((B, 128), f32),
        pltpu.VMEM((B, 128), f32),
        pltpu.VMEM((B // 8, 128), f32),
        pltpu.VMEM((NPB, 128), f32),
        pltpu.VMEM((NPB, 128), f32),
        pltpu.VMEM((NPB, 128), f32),
        pltpu.VMEM((NPB, 128), f32),
        pltpu.VMEM((NPB, 128), f32),
        pltpu.VMEM((32, 128), jnp.int32),
        pltpu.SemaphoreType.DMA,
    ]

    @functools.partial(pl.kernel, mesh=mesh, out_type=out_type,
                       scratch_types=scratch)
    def k(qlo, qhi, tlo, thi, klo, khi, vlo, vhi, eat, dsh, ssh, pmh, ebnd,
          ovlo, ovhi, oelo, oehi, p2, den2,
          dsw, ssw, pmw, g0, g1, g2, g3, g4, g5, g6, vb0, vb1, eb6, prow, den,
          a0, a1, a2, a3, ebrow, sem):
        w = lax.axis_index("s") * NC + lax.axis_index("c")
        pltpu.sync_copy(ebnd.at[w], ebrow)
        io16 = lax.iota(jnp.int32, 16)
        z16 = jnp.zeros((16,), f32)

        def bucket(jb, cb):
            ebv = ebrow[jb, pl.ds(0, 16)]
            e_lo = ebv[0]
            e_hi = ebv[1]
            n_base = pl.multiple_of((w * BPW + jb) * NPB, NPB)
            e_al = pl.multiple_of((e_lo // 64) * 64, 64)
            nwin = (e_hi - e_al + (B - 1)) // B

            def zbody(r, c):
                den[r, pl.ds(0, 16)] = z16
                for cc in range(8):
                    a0[r, pl.ds(cc * 16, 16)] = z16
                    a1[r, pl.ds(cc * 16, 16)] = z16
                    a2[r, pl.ds(cc * 16, 16)] = z16
                    a3[r, pl.ds(cc * 16, 16)] = z16
                return c
            lax.fori_loop(0, NPB, zbody, 0)

            def winA(it, carry):
                ew = pl.multiple_of(e_al + it * B, 64)
                pltpu.sync_copy(dsh.at[pl.ds(ew, B)], dsw)
                pltpu.sync_copy(ssh.at[pl.ds(ew, B)], ssw)
                pltpu.sync_copy(pmh.at[pl.ds(ew, B)], pmw)
                cps = [pltpu.async_copy(qlo.at[dsw], g0, sem),
                       pltpu.async_copy(qhi.at[dsw], g1, sem),
                       pltpu.async_copy(tlo.at[dsw], g2, sem),
                       pltpu.async_copy(thi.at[dsw], g3, sem),
                       pltpu.async_copy(klo.at[ssw], g4, sem),
                       pltpu.async_copy(khi.at[ssw], g5, sem),
                       pltpu.async_copy(eat.at[pmw], g6, sem)]
                for cp in cps:
                    cp.wait()

                def grpA(g, c2):
                    dsv = dsw[pl.ds(g * 16, 16)]
                    for j in range(16):
                        ei = g * 16 + j
                        gidx = ew + ei
                        d = dsv[j]
                        dl = jnp.clip(d - n_base, 0, NPB - 1)
                        inb = jnp.where((gidx >= e_lo) & (gidx < e_hi),
                                        f32(1.0), f32(0.0))
                        lsum = []
                        for h in range(H):
                            part = None
                            for cc in range(4):
                                gc = h * 4 + cc
                                qb = g0 if gc < 8 else g1
                                kb = g4 if gc < 8 else g5
                                col = (gc % 8) * 16
                                qv = qb[ei, pl.ds(col, 16)]
                                kv = kb[ei, pl.ds(col, 16)]
                                pr = qv * kv
                                part = pr if part is None else part + pr
                            for fc in range(4):
                                gc = h * 4 + fc
                                tb = g2 if gc < 8 else g3
                                col = (gc % 8) * 16
                                tv = tb[ei, pl.ds(col, 16)]
                                eav = g6[ei, pl.ds(fc * 16, 16)]
                                part = part + tv * eav
                            acc = [part[lane] + part[lane + 8]
                                   for lane in range(8)]
                            acc = [acc[lane] + acc[lane + 4]
                                   for lane in range(4)]
                            lsum.append((acc[0] + acc[1]) + (acc[2] + acc[3]))
                        lvec = z16
                        for h in range(H):
                            lvec = jnp.where(io16 == h,
                                             jnp.full((16,), lsum[h], f32),
                                             lvec)
                        pe = jnp.exp(lvec)
                        prow[g * 2 + j // 8, pl.ds((j % 8) * 16, 16)] = pe
                        den[dl, pl.ds(0, 16)] = (
                            den[dl, pl.ds(0, 16)]
                            + pe * jnp.full((16,), inb, f32))
                    return c2
                lax.fori_loop(0, B // 16, grpA, 0)
                pltpu.sync_copy(
                    prow, p2.at[pl.ds(pl.multiple_of(ew // 8, 8), B // 8)])
                return carry
            lax.fori_loop(0, nwin, winA, 0)
            pltpu.sync_copy(den, den2.at[pl.ds(n_base, NPB)])

            def winB(it, carry):
                ew = pl.multiple_of(e_al + it * B, 64)
                pltpu.sync_copy(dsh.at[pl.ds(ew, B)], dsw)
                pltpu.sync_copy(ssh.at[pl.ds(ew, B)], ssw)
                pltpu.sync_copy(pmh.at[pl.ds(ew, B)], pmw)
                pltpu.sync_copy(
                    p2.at[pl.ds(pl.multiple_of(ew // 8, 8), B // 8)], prow)
                cps = [pltpu.async_copy(vlo.at[ssw], vb0, sem),
                       pltpu.async_copy(vhi.at[ssw], vb1, sem),
                       pltpu.async_copy(eat.at[pmw], eb6, sem)]
                for cp in cps:
                    cp.wait()

                def grpB(g, c2):
                    dsv = dsw[pl.ds(g * 16, 16)]
                    for j in range(16):
                        ei = g * 16 + j
                        gidx = ew + ei
                        d = dsv[j]
                        dl = jnp.clip(d - n_base, 0, NPB - 1)
                        inb = jnp.where((gidx >= e_lo) & (gidx < e_hi),
                                        f32(1.0), f32(0.0))
                        prw = prow[g * 2 + j // 8, pl.ds((j % 8) * 16, 16)]
                        avec = prw * jnp.full((16,), inb, f32)
                        for h in range(H):
                            ahv = jnp.full((16,), avec[h], f32)
                            for cc in range(4):
                                gc = h * 4 + cc
                                vb = vb0 if gc < 8 else vb1
                                ab = a0 if gc < 8 else a1
                                col = (gc % 8) * 16
                                ab[dl, pl.ds(col, 16)] = (
                                    ab[dl, pl.ds(col, 16)]
                                    + ahv * vb[ei, pl.ds(col, 16)])
                            for fc in range(4):
                                gc = h * 4 + fc
                                ae = a2 if gc < 8 else a3
                                col = (gc % 8) * 16
                                ae[dl, pl.ds(col, 16)] = (
                                    ae[dl, pl.ds(col, 16)]
                                    + ahv * eb6[ei, pl.ds(fc * 16, 16)])
                    return c2
                lax.fori_loop(0, B // 16, grpB, 0)
                return carry
            lax.fori_loop(0, nwin, winB, 0)

            pltpu.sync_copy(a0, ovlo.at[pl.ds(n_base, NPB)])
            pltpu.sync_copy(a1, ovhi.at[pl.ds(n_base, NPB)])
            pltpu.sync_copy(a2, oelo.at[pl.ds(n_base, NPB)])
            pltpu.sync_copy(a3, oehi.at[pl.ds(n_base, NPB)])
            return cb

        lax.fori_loop(0, BPW, bucket, 0)

    return k


_SC_EDGE = _sc_edge_kernel()


def _embed_node_kernel(x, w, b):
    def body(x_ref, w_ref, b_ref, o_ref):
        o_ref[...] = jnp.dot(x_ref[...], w_ref[...],
                             preferred_element_type=jnp.float32) + b_ref[...]
    return pl.pallas_call(
        body,
        grid=(NP // BN,),
        in_specs=[pl.BlockSpec((BN, 87), lambda i: (i, 0)),
                  pl.BlockSpec((87, NEU), lambda i: (0, 0)),
                  pl.BlockSpec((1, NEU), lambda i: (0, 0))],
        out_specs=pl.BlockSpec((BN, NEU), lambda i: (i, 0)),
        out_shape=jax.ShapeDtypeStruct((NP, NEU), jnp.float32),
    )(x, w, b.reshape(1, NEU))


def _embed_edge_kernel(ea, w, b):
    def body(e_ref, w_ref, b_ref, o_ref):
        y = jnp.dot(e_ref[...], w_ref[...],
                    preferred_element_type=jnp.float32) + b_ref[...]
        y = jnp.where(y > 0, y, 0.2 * (jnp.exp(y) - 1.0))
        o_ref[...] = jnp.concatenate(
            [y, jnp.zeros((BE, 128 - NEU), jnp.float32)], axis=1)
    return pl.pallas_call(
        body,
        grid=(E // BE,),
        in_specs=[pl.BlockSpec((BE, 41), lambda i: (i, 0)),
                  pl.BlockSpec((41, NEU), lambda i: (0, 0)),
                  pl.BlockSpec((1, NEU), lambda i: (0, 0))],
        out_specs=pl.BlockSpec((BE, 128), lambda i: (i, 0)),
        out_shape=jax.ShapeDtypeStruct((E, 128), jnp.float32),
    )(ea, w, b.reshape(1, NEU))


def _proj_kernel(h, wall, ball):
    def body(h_ref, w_ref, b_ref, *outs):
        y = jnp.dot(h_ref[...], w_ref[...],
                    preferred_element_type=jnp.float32) + b_ref[...]
        for t in range(8):
            outs[t][...] = y[:, t * 128:(t + 1) * 128]
    spec128 = pl.BlockSpec((BN, 128), lambda i: (i, 0))
    return pl.pallas_call(
        body,
        grid=(NP // BN,),
        in_specs=[pl.BlockSpec((BN, NEU), lambda i: (i, 0)),
                  pl.BlockSpec((NEU, 1024), lambda i: (0, 0)),
                  pl.BlockSpec((1, 1024), lambda i: (0, 0))],
        out_specs=[spec128] * 8,
        out_shape=[jax.ShapeDtypeStruct((NP, 128), jnp.float32)] * 8,
    )(h, wall, ball.reshape(1, 1024))


def _epilogue_kernel(ovlo, ovhi, oelo, oehi, den2, h, webd, wskip, bskip,
                     wlin, blin):
    nsteps = NP // BN

    def body(vlo_r, vhi_r, elo_r, ehi_r, dn_r, h_r, webd_r, wsk_r, bsk_r,
             wlin_r, blin_r, hn_r, st_r):
        i = pl.program_id(0)
        outv = jnp.concatenate([vlo_r[...], vhi_r[...]], axis=1)
        outea = jnp.concatenate([elo_r[...], ehi_r[...]], axis=1)
        den4 = dn_r[...][:, :H]
        hsel = (lax.broadcasted_iota(jnp.int32, (H, H * C), 0)
                == lax.broadcasted_iota(jnp.int32, (H, H * C), 1) // C
                ).astype(jnp.float32)
        div = jnp.dot(den4, hsel,
                      preferred_element_type=jnp.float32) + 1e-16
        outv = outv / div
        outea = outea / div
        z = (outv
             + jnp.dot(outea, webd_r[...], preferred_element_type=jnp.float32)
             + jnp.dot(h_r[...], wsk_r[...], preferred_element_type=jnp.float32)
             + bsk_r[...])
        g = jnp.dot(z, wlin_r[...], preferred_element_type=jnp.float32) \
            + blin_r[...]
        hn = jnp.where(g > 0, g, jnp.exp(g) - 1.0)
        hn_r[...] = hn
        rows = i * BN + lax.broadcasted_iota(jnp.int32, (BN, 1), 0)
        msk = jnp.where(rows < N, 1.0, 0.0).astype(jnp.float32)
        hm = hn * msk
        s1 = jnp.sum(hm, axis=0)
        s2 = jnp.sum(hm * hm, axis=0)
        pad = jnp.zeros((128 - NEU,), jnp.float32)
        srow = jnp.concatenate([s1, pad])[None, :]
        qrow = jnp.concatenate([s2, pad])[None, :]
        blk = jnp.concatenate(
            [srow, qrow, jnp.zeros((6, 128), jnp.float32)], axis=0)

        @pl.when(i == 0)
        def _():
            st_r[...] = blk

        @pl.when(i > 0)
        def _():
            st_r[...] = st_r[...] + blk

    spec128 = pl.BlockSpec((BN, 128), lambda i: (i, 0))
    return pl.pallas_call(
        body,
        grid=(nsteps,),
        in_specs=[spec128, spec128, spec128, spec128, spec128,
                  pl.BlockSpec((BN, NEU), lambda i: (i, 0)),
                  pl.BlockSpec((256, 256), lambda i: (0, 0)),
                  pl.BlockSpec((NEU, 256), lambda i: (0, 0)),
                  pl.BlockSpec((1, 256), lambda i: (0, 0)),
                  pl.BlockSpec((256, NEU), lambda i: (0, 0)),
                  pl.BlockSpec((1, NEU), lambda i: (0, 0))],
        out_specs=[pl.BlockSpec((BN, NEU), lambda i: (i, 0)),
                   pl.BlockSpec((8, 128), lambda i: (0, 0))],
        out_shape=[jax.ShapeDtypeStruct((NP, NEU), jnp.float32),
                   jax.ShapeDtypeStruct((8, 128), jnp.float32)],
    )(ovlo, ovhi, oelo, oehi, den2, h, webd, wskip, bskip.reshape(1, 256),
      wlin, blin.reshape(1, NEU))


def _bn(hn, st, gamma, beta):
    mean = st[0, :NEU] / N
    var = st[1, :NEU] / N - mean * mean
    hb = gamma * (hn - mean) / jnp.sqrt(var + 1e-5) + beta
    return jnp.maximum(hb, 0.0)


def _bnproj_kernel(hn, stats, gamma, beta, wall, ball):
    def body(hn_r, st_r, g_r, b_r, w_ref, bl_ref, h_out, *outs):
        h = _bn(hn_r[...], st_r[...], g_r[...], b_r[...])
        h_out[...] = h
        y = jnp.dot(h, w_ref[...],
                    preferred_element_type=jnp.float32) + bl_ref[...]
        for t in range(8):
            outs[t][...] = y[:, t * 128:(t + 1) * 128]
    spec128 = pl.BlockSpec((BN, 128), lambda i: (i, 0))
    return pl.pallas_call(
        body,
        grid=(NP // BN,),
        in_specs=[pl.BlockSpec((BN, NEU), lambda i: (i, 0)),
                  pl.BlockSpec((8, 128), lambda i: (0, 0)),
                  pl.BlockSpec((1, NEU), lambda i: (0, 0)),
                  pl.BlockSpec((1, NEU), lambda i: (0, 0)),
                  pl.BlockSpec((NEU, 1024), lambda i: (0, 0)),
                  pl.BlockSpec((1, 1024), lambda i: (0, 0))],
        out_specs=[pl.BlockSpec((BN, NEU), lambda i: (i, 0))] + [spec128] * 8,
        out_shape=[jax.ShapeDtypeStruct((NP, NEU), jnp.float32)]
        + [jax.ShapeDtypeStruct((NP, 128), jnp.float32)] * 8,
    )(hn, stats, gamma.reshape(1, NEU), beta.reshape(1, NEU), wall,
      ball.reshape(1, 1024))


def _pool_kernel(hn, stats, gamma, beta, batch2d, wfin, bfin):
    nsteps = NP // BN

    def body(hn_r, st_r, g_r, b_r, bt_r, wf_r, bf_r, o_ref, acc_ref):
        i = pl.program_id(0)
        h = _bn(hn_r[...], st_r[...], g_r[...], b_r[...])
        seg = bt_r[...].reshape(1, BN)
        gi = lax.broadcasted_iota(jnp.int32, (G, BN), 0)
        oh = jnp.where(gi == seg, 1.0, 0.0).astype(jnp.float32)
        hp = jnp.concatenate(
            [h, jnp.ones((BN, 1), jnp.float32),
             jnp.zeros((BN, 128 - NEU - 1), jnp.float32)], axis=1)
        blk = jnp.dot(oh, hp, preferred_element_type=jnp.float32)

        @pl.when(i == 0)
        def _():
            acc_ref[...] = blk

        @pl.when(i > 0)
        def _():
            acc_ref[...] = acc_ref[...] + blk

        @pl.when(i == nsteps - 1)
        def _():
            tot = acc_ref[...] if nsteps > 1 else blk
            cnt = jnp.maximum(tot[:, NEU:NEU + 1], 1.0)
            pooled = tot[:, :NEU] / cnt
            o_ref[...] = jnp.dot(pooled, wf_r[...],
                                 preferred_element_type=jnp.float32) + bf_r[...]

    return pl.pallas_call(
        body,
        grid=(nsteps,),
        in_specs=[pl.BlockSpec((BN, NEU), lambda i: (i, 0)),
                  pl.BlockSpec((8, 128), lambda i: (0, 0)),
                  pl.BlockSpec((1, NEU), lambda i: (0, 0)),
                  pl.BlockSpec((1, NEU), lambda i: (0, 0)),
                  pl.BlockSpec((1, 1, BN), lambda i: (i, 0, 0)),
                  pl.BlockSpec((NEU, NEU), lambda i: (0, 0)),
                  pl.BlockSpec((1, NEU), lambda i: (0, 0))],
        out_specs=[pl.BlockSpec((G, NEU), lambda i: (0, 0)),
                   pl.BlockSpec((G, 128), lambda i: (0, 0))],
        out_shape=[jax.ShapeDtypeStruct((G, NEU), jnp.float32),
                   jax.ShapeDtypeStruct((G, 128), jnp.float32)],
    )(hn, stats, gamma.reshape(1, NEU), beta.reshape(1, NEU), batch2d,
      wfin, bfin.reshape(1, NEU))[0]


def _layer_weights(lp):
    s = 1.0 / jnp.sqrt(jnp.float32(C))
    wq = lp['Wq'] * s
    bq = lp['bq'] * s
    we4 = lp['We'].reshape(NEU, H, C)
    wq4 = wq.reshape(NEU, H, C)
    wt = jnp.einsum('ihc,fhc->ihf', wq4, we4).reshape(NEU, H * C)
    bt = jnp.einsum('hc,fhc->hf', bq.reshape(H, C), we4).reshape(H * C)
    wall = jnp.concatenate([wq, wt, lp['Wk'], lp['Wv']], axis=1)
    ball = jnp.concatenate([bq, bt, lp['bk'], lp['bv']])
    webd = jax.scipy.linalg.block_diag(
        *[we4[:, h, :] for h in range(H)])
    return wall, ball, webd


def kernel(x, edge_index, edge_attr, batch, params):
    f32 = jnp.float32
    src = edge_index[0].astype(jnp.int32)
    dst = edge_index[1].astype(jnp.int32)

    perm = jnp.argsort(dst).astype(jnp.int32)
    ds = jnp.take(dst, perm)
    ss = jnp.take(src, perm)
    starts = (jnp.arange(NBUCKET + 1, dtype=jnp.int32) * NPB)
    ebnd = jnp.searchsorted(ds, starts).astype(jnp.int32)
    bidx = (jnp.arange(NW, dtype=jnp.int32)[:, None, None] * BPW
            + jnp.arange(32, dtype=jnp.int32)[None, :, None]
            + jnp.arange(2, dtype=jnp.int32)[None, None, :]).clip(0, NBUCKET)
    ebnd_m = jnp.take(ebnd, bidx)
    ebnd_m = jnp.pad(ebnd_m, ((0, 0), (0, 0), (0, 126)))
    ds_p = jnp.pad(ds, (0, EPAD - E))
    ss_p = jnp.pad(ss, (0, EPAD - E))
    pm_p = jnp.pad(perm, (0, EPAD - E))

    x_p = jnp.pad(x, ((0, NP - N), (0, 0)))
    batch_p = jnp.pad(batch.astype(jnp.int32), (0, NP - N),
                      constant_values=G).reshape(NP // BN, 1, BN)

    h = _embed_node_kernel(x_p, params['embed_n_W'],
                           params['embed_n_b'].astype(f32))
    eat = _embed_edge_kernel(edge_attr, params['embed_e_W'],
                             params['embed_e_b'].astype(f32))

    lws = [_layer_weights(lp) for lp in params['layers']]

    hn, stats = None, None
    for li, lp in enumerate(params['layers']):
        wall, ball, webd = lws[li]
        if li == 0:
            tabs = _proj_kernel(h, wall, ball)
        else:
            h, *tabs = _bnproj_kernel(hn, stats, lp_prev['gamma'],
                                      lp_prev['beta'], wall, ball)
        qlo, qhi, tlo, thi, klo, khi, vlo, vhi = tabs
        ovlo, ovhi, oelo, oehi, _p2, den2 = _SC_EDGE(
            qlo, qhi, tlo, thi, klo, khi, vlo, vhi, eat,
            ds_p, ss_p, pm_p, ebnd_m)
        hn, stats = _epilogue_kernel(ovlo, ovhi, oelo, oehi, den2, h, webd,
                                     lp['Wskip'], lp['bskip'],
                                     lp['Wlin'], lp['blin'])
        lp_prev = lp

    return _pool_kernel(hn, stats, lp_prev['gamma'], lp_prev['beta'],
                        batch_p, params['lin3_W'], params['lin3_b'])


# speedup vs baseline: 6.6940x; 1.2705x over previous
import functools

import jax
import jax.numpy as jnp
from jax import lax
from jax.experimental import pallas as pl
from jax.experimental.pallas import tpu as pltpu, tpu_sc as plsc

N = 50000
E = 800000
G = 128
H = 4
C = 64
NEU = 64

NC, NS = 2, 16
NW = NC * NS
NPB = 64
NBUCKET = 800
BPW = NBUCKET // NW
NP = NBUCKET * NPB
B = 64
EPAD = E + 2 * B
BN = 1024
BE = 2000


def _sc_edge_kernel():
    mesh = plsc.VectorSubcoreMesh(core_axis_name="c", subcore_axis_name="s")
    f32 = jnp.float32
    out_type = [
        jax.ShapeDtypeStruct((NP, 128), f32),
        jax.ShapeDtypeStruct((NP, 128), f32),
        jax.ShapeDtypeStruct((NP, 128), f32),
        jax.ShapeDtypeStruct((NP, 128), f32),
        jax.ShapeDtypeStruct((EPAD // 8, 128), f32),
        jax.ShapeDtypeStruct((NP, 128), f32),
    ]
    scratch = [
        pltpu.VMEM((B,), jnp.int32),
        pltpu.VMEM((B,), jnp.int32),
        pltpu.VMEM((B,), jnp.int32),
        pltpu.VMEM((B, 128), f32),
        pltpu.VMEM((B, 128), f32),
        pltpu.VMEM((B, 128), f32),
        pltpu.VMEM((B, 128), f32),
        pltpu.VMEM((B, 128), f32),
        pltpu.VMEM((B, 128), f32),
        pltpu.VMEM((B, 128), f32),
        pltpu.VMEM((B, 128), f32),
        pltpu.VMEM((B, 128), f32),
        pltpu.VMEM((B, 128), f32),
        pltpu.VMEM((B // 8, 128), f32),
        pltpu.VMEM((NPB, 128), f32),
        pltpu.VMEM((NPB, 128), f32),
        pltpu.VMEM((NPB, 128), f32),
        pltpu.VMEM((NPB, 128), f32),
        pltpu.VMEM((NPB, 128), f32),
        pltpu.VMEM((32, 128), jnp.int32),
        pltpu.SemaphoreType.DMA,
    ]

    @functools.partial(pl.kernel, mesh=mesh, out_type=out_type,
                       scratch_types=scratch)
    def k(qlo, qhi, tlo, thi, klo, khi, vlo, vhi, eat, dsh, ssh, pmh, ebnd,
          ovlo, ovhi, oelo, oehi, p2, den2,
          dsw, ssw, pmw, g0, g1, g2, g3, g4, g5, g6, vb0, vb1, eb6, prow, den,
          a0, a1, a2, a3, ebrow, sem):
        w = lax.axis_index("s") * NC + lax.axis_index("c")
        pltpu.sync_copy(ebnd.at[w], ebrow)
        io16 = lax.iota(jnp.int32, 16)
        z16 = jnp.zeros((16,), f32)

        def bucket(jb, cb):
            ebv = ebrow[jb, pl.ds(0, 16)]
            e_lo = ebv[0]
            e_hi = ebv[1]
            n_base = pl.multiple_of((w * BPW + jb) * NPB, NPB)
            e_al = pl.multiple_of((e_lo // 64) * 64, 64)
            nwin = (e_hi - e_al + (B - 1)) // B

            def zbody(r, c):
                den[r, pl.ds(0, 16)] = z16
                for cc in range(8):
                    a0[r, pl.ds(cc * 16, 16)] = z16
                    a1[r, pl.ds(cc * 16, 16)] = z16
                    a2[r, pl.ds(cc * 16, 16)] = z16
                    a3[r, pl.ds(cc * 16, 16)] = z16
                return c
            lax.fori_loop(0, NPB, zbody, 0)

            pltpu.sync_copy(qlo.at[pl.ds(n_base, NPB)], g0)
            pltpu.sync_copy(qhi.at[pl.ds(n_base, NPB)], g1)
            pltpu.sync_copy(tlo.at[pl.ds(n_base, NPB)], g2)
            pltpu.sync_copy(thi.at[pl.ds(n_base, NPB)], g3)

            def win(it, carry):
                ew = pl.multiple_of(e_al + it * B, 64)
                pltpu.sync_copy(dsh.at[pl.ds(ew, B)], dsw)
                pltpu.sync_copy(ssh.at[pl.ds(ew, B)], ssw)
                pltpu.sync_copy(pmh.at[pl.ds(ew, B)], pmw)
                cps = [pltpu.async_copy(klo.at[ssw], g4, sem),
                       pltpu.async_copy(khi.at[ssw], g5, sem),
                       pltpu.async_copy(vlo.at[ssw], vb0, sem),
                       pltpu.async_copy(vhi.at[ssw], vb1, sem),
                       pltpu.async_copy(eat.at[pmw], g6, sem)]
                for cp in cps:
                    cp.wait()

                def grp(g, c2):
                    dsv = dsw[pl.ds(g * 16, 16)]
                    for j in range(16):
                        ei = g * 16 + j
                        gidx = ew + ei
                        d = dsv[j]
                        dl = jnp.clip(d - n_base, 0, NPB - 1)
                        inb = jnp.where((gidx >= e_lo) & (gidx < e_hi),
                                        f32(1.0), f32(0.0))
                        lsum = []
                        for h in range(H):
                            part = None
                            for cc in range(4):
                                gc = h * 4 + cc
                                qb = g0 if gc < 8 else g1
                                kb = g4 if gc < 8 else g5
                                col = (gc % 8) * 16
                                qv = qb[dl, pl.ds(col, 16)]
                                kv = kb[ei, pl.ds(col, 16)]
                                pr = qv * kv
                                part = pr if part is None else part + pr
                            for fc in range(4):
                                gc = h * 4 + fc
                                tb = g2 if gc < 8 else g3
                                col = (gc % 8) * 16
                                tv = tb[dl, pl.ds(col, 16)]
                                eav = g6[ei, pl.ds(fc * 16, 16)]
                                part = part + tv * eav
                            acc = [part[lane] + part[lane + 8]
                                   for lane in range(8)]
                            acc = [acc[lane] + acc[lane + 4]
                                   for lane in range(4)]
                            lsum.append((acc[0] + acc[1]) + (acc[2] + acc[3]))
                        lvec = z16
                        for h in range(H):
                            lvec = jnp.where(io16 == h,
                                             jnp.full((16,), lsum[h], f32),
                                             lvec)
                        avec = jnp.exp(lvec) * jnp.full((16,), inb, f32)
                        den[dl, pl.ds(0, 16)] = (
                            den[dl, pl.ds(0, 16)] + avec)
                        for h in range(H):
                            ahv = jnp.full((16,), avec[h], f32)
                            for cc in range(4):
                                gc = h * 4 + cc
                                vb = vb0 if gc < 8 else vb1
                                ab = a0 if gc < 8 else a1
                                col = (gc % 8) * 16
                                ab[dl, pl.ds(col, 16)] = (
                                    ab[dl, pl.ds(col, 16)]
                                    + ahv * vb[ei, pl.ds(col, 16)])
                            for fc in range(4):
                                gc = h * 4 + fc
                                ae = a2 if gc < 8 else a3
                                col = (gc % 8) * 16
                                ae[dl, pl.ds(col, 16)] = (
                                    ae[dl, pl.ds(col, 16)]
                                    + ahv * g6[ei, pl.ds(fc * 16, 16)])
                    return c2
                lax.fori_loop(0, B // 16, grp, 0)
                return carry
            lax.fori_loop(0, nwin, win, 0)
            pltpu.sync_copy(den, den2.at[pl.ds(n_base, NPB)])

            pltpu.sync_copy(a0, ovlo.at[pl.ds(n_base, NPB)])
            pltpu.sync_copy(a1, ovhi.at[pl.ds(n_base, NPB)])
            pltpu.sync_copy(a2, oelo.at[pl.ds(n_base, NPB)])
            pltpu.sync_copy(a3, oehi.at[pl.ds(n_base, NPB)])
            return cb

        lax.fori_loop(0, BPW, bucket, 0)

    return k


_SC_EDGE = _sc_edge_kernel()


def _embed_node_kernel(x, w, b):
    def body(x_ref, w_ref, b_ref, o_ref):
        o_ref[...] = jnp.dot(x_ref[...], w_ref[...],
                             preferred_element_type=jnp.float32) + b_ref[...]
    return pl.pallas_call(
        body,
        grid=(NP // BN,),
        in_specs=[pl.BlockSpec((BN, 87), lambda i: (i, 0)),
                  pl.BlockSpec((87, NEU), lambda i: (0, 0)),
                  pl.BlockSpec((1, NEU), lambda i: (0, 0))],
        out_specs=pl.BlockSpec((BN, NEU), lambda i: (i, 0)),
        out_shape=jax.ShapeDtypeStruct((NP, NEU), jnp.float32),
    )(x, w, b.reshape(1, NEU))


def _embed_edge_kernel(ea, w, b):
    def body(e_ref, w_ref, b_ref, o_ref):
        y = jnp.dot(e_ref[...], w_ref[...],
                    preferred_element_type=jnp.float32) + b_ref[...]
        y = jnp.where(y > 0, y, 0.2 * (jnp.exp(y) - 1.0))
        o_ref[...] = jnp.concatenate(
            [y, jnp.zeros((BE, 128 - NEU), jnp.float32)], axis=1)
    return pl.pallas_call(
        body,
        grid=(E // BE,),
        in_specs=[pl.BlockSpec((BE, 41), lambda i: (i, 0)),
                  pl.BlockSpec((41, NEU), lambda i: (0, 0)),
                  pl.BlockSpec((1, NEU), lambda i: (0, 0))],
        out_specs=pl.BlockSpec((BE, 128), lambda i: (i, 0)),
        out_shape=jax.ShapeDtypeStruct((E, 128), jnp.float32),
    )(ea, w, b.reshape(1, NEU))


def _proj_kernel(h, wall, ball):
    def body(h_ref, w_ref, b_ref, *outs):
        y = jnp.dot(h_ref[...], w_ref[...],
                    preferred_element_type=jnp.float32) + b_ref[...]
        for t in range(8):
            outs[t][...] = y[:, t * 128:(t + 1) * 128]
    spec128 = pl.BlockSpec((BN, 128), lambda i: (i, 0))
    return pl.pallas_call(
        body,
        grid=(NP // BN,),
        in_specs=[pl.BlockSpec((BN, NEU), lambda i: (i, 0)),
                  pl.BlockSpec((NEU, 1024), lambda i: (0, 0)),
                  pl.BlockSpec((1, 1024), lambda i: (0, 0))],
        out_specs=[spec128] * 8,
        out_shape=[jax.ShapeDtypeStruct((NP, 128), jnp.float32)] * 8,
    )(h, wall, ball.reshape(1, 1024))


def _epilogue_kernel(ovlo, ovhi, oelo, oehi, den2, h, webd, wskip, bskip,
                     wlin, blin):
    nsteps = NP // BN

    def body(vlo_r, vhi_r, elo_r, ehi_r, dn_r, h_r, webd_r, wsk_r, bsk_r,
             wlin_r, blin_r, hn_r, st_r):
        i = pl.program_id(0)
        outv = jnp.concatenate([vlo_r[...], vhi_r[...]], axis=1)
        outea = jnp.concatenate([elo_r[...], ehi_r[...]], axis=1)
        den4 = dn_r[...][:, :H]
        hsel = (lax.broadcasted_iota(jnp.int32, (H, H * C), 0)
                == lax.broadcasted_iota(jnp.int32, (H, H * C), 1) // C
                ).astype(jnp.float32)
        div = jnp.dot(den4, hsel,
                      preferred_element_type=jnp.float32) + 1e-16
        outv = outv / div
        outea = outea / div
        z = (outv
             + jnp.dot(outea, webd_r[...], preferred_element_type=jnp.float32)
             + jnp.dot(h_r[...], wsk_r[...], preferred_element_type=jnp.float32)
             + bsk_r[...])
        g = jnp.dot(z, wlin_r[...], preferred_element_type=jnp.float32) \
            + blin_r[...]
        hn = jnp.where(g > 0, g, jnp.exp(g) - 1.0)
        hn_r[...] = hn
        rows = i * BN + lax.broadcasted_iota(jnp.int32, (BN, 1), 0)
        msk = jnp.where(rows < N, 1.0, 0.0).astype(jnp.float32)
        hm = hn * msk
        s1 = jnp.sum(hm, axis=0)
        s2 = jnp.sum(hm * hm, axis=0)
        pad = jnp.zeros((128 - NEU,), jnp.float32)
        srow = jnp.concatenate([s1, pad])[None, :]
        qrow = jnp.concatenate([s2, pad])[None, :]
        blk = jnp.concatenate(
            [srow, qrow, jnp.zeros((6, 128), jnp.float32)], axis=0)

        @pl.when(i == 0)
        def _():
            st_r[...] = blk

        @pl.when(i > 0)
        def _():
            st_r[...] = st_r[...] + blk

    spec128 = pl.BlockSpec((BN, 128), lambda i: (i, 0))
    return pl.pallas_call(
        body,
        grid=(nsteps,),
        in_specs=[spec128, spec128, spec128, spec128, spec128,
                  pl.BlockSpec((BN, NEU), lambda i: (i, 0)),
                  pl.BlockSpec((256, 256), lambda i: (0, 0)),
                  pl.BlockSpec((NEU, 256), lambda i: (0, 0)),
                  pl.BlockSpec((1, 256), lambda i: (0, 0)),
                  pl.BlockSpec((256, NEU), lambda i: (0, 0)),
                  pl.BlockSpec((1, NEU), lambda i: (0, 0))],
        out_specs=[pl.BlockSpec((BN, NEU), lambda i: (i, 0)),
                   pl.BlockSpec((8, 128), lambda i: (0, 0))],
        out_shape=[jax.ShapeDtypeStruct((NP, NEU), jnp.float32),
                   jax.ShapeDtypeStruct((8, 128), jnp.float32)],
    )(ovlo, ovhi, oelo, oehi, den2, h, webd, wskip, bskip.reshape(1, 256),
      wlin, blin.reshape(1, NEU))


def _bn(hn, st, gamma, beta):
    mean = st[0, :NEU] / N
    var = st[1, :NEU] / N - mean * mean
    hb = gamma * (hn - mean) / jnp.sqrt(var + 1e-5) + beta
    return jnp.maximum(hb, 0.0)


def _bnproj_kernel(hn, stats, gamma, beta, wall, ball):
    def body(hn_r, st_r, g_r, b_r, w_ref, bl_ref, h_out, *outs):
        h = _bn(hn_r[...], st_r[...], g_r[...], b_r[...])
        h_out[...] = h
        y = jnp.dot(h, w_ref[...],
                    preferred_element_type=jnp.float32) + bl_ref[...]
        for t in range(8):
            outs[t][...] = y[:, t * 128:(t + 1) * 128]
    spec128 = pl.BlockSpec((BN, 128), lambda i: (i, 0))
    return pl.pallas_call(
        body,
        grid=(NP // BN,),
        in_specs=[pl.BlockSpec((BN, NEU), lambda i: (i, 0)),
                  pl.BlockSpec((8, 128), lambda i: (0, 0)),
                  pl.BlockSpec((1, NEU), lambda i: (0, 0)),
                  pl.BlockSpec((1, NEU), lambda i: (0, 0)),
                  pl.BlockSpec((NEU, 1024), lambda i: (0, 0)),
                  pl.BlockSpec((1, 1024), lambda i: (0, 0))],
        out_specs=[pl.BlockSpec((BN, NEU), lambda i: (i, 0))] + [spec128] * 8,
        out_shape=[jax.ShapeDtypeStruct((NP, NEU), jnp.float32)]
        + [jax.ShapeDtypeStruct((NP, 128), jnp.float32)] * 8,
    )(hn, stats, gamma.reshape(1, NEU), beta.reshape(1, NEU), wall,
      ball.reshape(1, 1024))


def _pool_kernel(hn, stats, gamma, beta, batch2d, wfin, bfin):
    nsteps = NP // BN

    def body(hn_r, st_r, g_r, b_r, bt_r, wf_r, bf_r, o_ref, acc_ref):
        i = pl.program_id(0)
        h = _bn(hn_r[...], st_r[...], g_r[...], b_r[...])
        seg = bt_r[...].reshape(1, BN)
        gi = lax.broadcasted_iota(jnp.int32, (G, BN), 0)
        oh = jnp.where(gi == seg, 1.0, 0.0).astype(jnp.float32)
        hp = jnp.concatenate(
            [h, jnp.ones((BN, 1), jnp.float32),
             jnp.zeros((BN, 128 - NEU - 1), jnp.float32)], axis=1)
        blk = jnp.dot(oh, hp, preferred_element_type=jnp.float32)

        @pl.when(i == 0)
        def _():
            acc_ref[...] = blk

        @pl.when(i > 0)
        def _():
            acc_ref[...] = acc_ref[...] + blk

        @pl.when(i == nsteps - 1)
        def _():
            tot = acc_ref[...] if nsteps > 1 else blk
            cnt = jnp.maximum(tot[:, NEU:NEU + 1], 1.0)
            pooled = tot[:, :NEU] / cnt
            o_ref[...] = jnp.dot(pooled, wf_r[...],
                                 preferred_element_type=jnp.float32) + bf_r[...]

    return pl.pallas_call(
        body,
        grid=(nsteps,),
        in_specs=[pl.BlockSpec((BN, NEU), lambda i: (i, 0)),
                  pl.BlockSpec((8, 128), lambda i: (0, 0)),
                  pl.BlockSpec((1, NEU), lambda i: (0, 0)),
                  pl.BlockSpec((1, NEU), lambda i: (0, 0)),
                  pl.BlockSpec((1, 1, BN), lambda i: (i, 0, 0)),
                  pl.BlockSpec((NEU, NEU), lambda i: (0, 0)),
                  pl.BlockSpec((1, NEU), lambda i: (0, 0))],
        out_specs=[pl.BlockSpec((G, NEU), lambda i: (0, 0)),
                   pl.BlockSpec((G, 128), lambda i: (0, 0))],
        out_shape=[jax.ShapeDtypeStruct((G, NEU), jnp.float32),
                   jax.ShapeDtypeStruct((G, 128), jnp.float32)],
    )(hn, stats, gamma.reshape(1, NEU), beta.reshape(1, NEU), batch2d,
      wfin, bfin.reshape(1, NEU))[0]


def _layer_weights(lp):
    s = 1.0 / jnp.sqrt(jnp.float32(C))
    wq = lp['Wq'] * s
    bq = lp['bq'] * s
    we4 = lp['We'].reshape(NEU, H, C)
    wq4 = wq.reshape(NEU, H, C)
    wt = jnp.einsum('ihc,fhc->ihf', wq4, we4).reshape(NEU, H * C)
    bt = jnp.einsum('hc,fhc->hf', bq.reshape(H, C), we4).reshape(H * C)
    wall = jnp.concatenate([wq, wt, lp['Wk'], lp['Wv']], axis=1)
    ball = jnp.concatenate([bq, bt, lp['bk'], lp['bv']])
    webd = jax.scipy.linalg.block_diag(
        *[we4[:, h, :] for h in range(H)])
    return wall, ball, webd


def kernel(x, edge_index, edge_attr, batch, params):
    f32 = jnp.float32
    src = edge_index[0].astype(jnp.int32)
    dst = edge_index[1].astype(jnp.int32)

    perm = jnp.argsort(dst).astype(jnp.int32)
    ds = jnp.take(dst, perm)
    ss = jnp.take(src, perm)
    starts = (jnp.arange(NBUCKET + 1, dtype=jnp.int32) * NPB)
    ebnd = jnp.searchsorted(ds, starts).astype(jnp.int32)
    bidx = (jnp.arange(NW, dtype=jnp.int32)[:, None, None] * BPW
            + jnp.arange(32, dtype=jnp.int32)[None, :, None]
            + jnp.arange(2, dtype=jnp.int32)[None, None, :]).clip(0, NBUCKET)
    ebnd_m = jnp.take(ebnd, bidx)
    ebnd_m = jnp.pad(ebnd_m, ((0, 0), (0, 0), (0, 126)))
    ds_p = jnp.pad(ds, (0, EPAD - E))
    ss_p = jnp.pad(ss, (0, EPAD - E))
    pm_p = jnp.pad(perm, (0, EPAD - E))

    x_p = jnp.pad(x, ((0, NP - N), (0, 0)))
    batch_p = jnp.pad(batch.astype(jnp.int32), (0, NP - N),
                      constant_values=G).reshape(NP // BN, 1, BN)

    h = _embed_node_kernel(x_p, params['embed_n_W'],
                           params['embed_n_b'].astype(f32))
    eat = _embed_edge_kernel(edge_attr, params['embed_e_W'],
                             params['embed_e_b'].astype(f32))

    lws = [_layer_weights(lp) for lp in params['layers']]

    hn, stats = None, None
    for li, lp in enumerate(params['layers']):
        wall, ball, webd = lws[li]
        if li == 0:
            tabs = _proj_kernel(h, wall, ball)
        else:
            h, *tabs = _bnproj_kernel(hn, stats, lp_prev['gamma'],
                                      lp_prev['beta'], wall, ball)
        qlo, qhi, tlo, thi, klo, khi, vlo, vhi = tabs
        ovlo, ovhi, oelo, oehi, _p2, den2 = _SC_EDGE(
            qlo, qhi, tlo, thi, klo, khi, vlo, vhi, eat,
            ds_p, ss_p, pm_p, ebnd_m)
        hn, stats = _epilogue_kernel(ovlo, ovhi, oelo, oehi, den2, h, webd,
                                     lp['Wskip'], lp['bskip'],
                                     lp['Wlin'], lp['blin'])
        lp_prev = lp

    return _pool_kernel(hn, stats, lp_prev['gamma'], lp_prev['beta'],
                        batch_p, params['lin3_W'], params['lin3_b'])
